# baseline (speedup 1.0000x reference)
import functools

import jax
import jax.numpy as jnp
from jax import lax
from jax.experimental import pallas as pl
from jax.experimental.pallas import tpu as pltpu
from jax.experimental.pallas import tpu_sc as plsc

N = 10000
D = 128
E = 320000
TAU0 = 0.5

NPAD = 10240
SLICE = 640
CHUNK = 128
SCROWS = 16
SUPER = 5
EPAD = 32 * SUPER * SCROWS * CHUNK
EROWS = EPAD // CHUNK
HBLK = 640


def _h_body(x_ref, w_ref, o_ref):
    i = pl.program_id(0)
    h = jnp.dot(x_ref[...], w_ref[...], preferred_element_type=jnp.float32)
    rows = lax.broadcasted_iota(jnp.int32, (HBLK, 1), 0) + i * HBLK
    o_ref[...] = jnp.where(rows < N, h, 0.0)


_h_matvec = pl.pallas_call(
    _h_body,
    grid=(NPAD // HBLK,),
    in_specs=[
        pl.BlockSpec((HBLK, D), lambda i: (i, jnp.int32(0))),
        pl.BlockSpec((D, 1), lambda i: (jnp.int32(0), jnp.int32(0))),
    ],
    out_specs=pl.BlockSpec((HBLK, 1), lambda i: (i, jnp.int32(0))),
    out_shape=jax.ShapeDtypeStruct((NPAD, 1), jnp.float32),
)


_mesh = plsc.VectorSubcoreMesh(core_axis_name="c", subcore_axis_name="s")


def _zero_vmem(ref, n):
    for j in range(n // 16):
        ref[pl.ds(j * 16, 16)] = jnp.zeros((16,), jnp.float32)


def _publish_and_reduce(hist, slots, comb, tmp, s, out_hbm, out_base):
    sl = s * jnp.int32(SLICE)
    pltpu.sync_copy(hist, slots.at[pl.ds(s * jnp.int32(NPAD), NPAD)])
    plsc.subcore_barrier()
    pltpu.sync_copy(slots.at[pl.ds(sl, SLICE)], comb)
    for t in range(1, 16):
        pltpu.sync_copy(slots.at[pl.ds(jnp.int32(t * NPAD) + sl, SLICE)], tmp)
        for j in range(SLICE // 16):
            dsl = pl.ds(j * 16, 16)
            comb[dsl] = comb[dsl] + tmp[dsl]
    pltpu.sync_copy(comb, out_hbm.at[pl.ds(out_base, SLICE)])


@functools.partial(
    pl.kernel,
    mesh=_mesh,
    out_type=jax.ShapeDtypeStruct((2 * NPAD,), jnp.float32),
    compiler_params=pltpu.CompilerParams(needs_layout_passes=False),
    scratch_types=[
        pltpu.VMEM((2 * SCROWS, CHUNK), jnp.int32),
        pltpu.VMEM((NPAD,), jnp.float32),
        pltpu.VMEM((SLICE,), jnp.float32),
        pltpu.VMEM((SLICE,), jnp.float32),
        pltpu.VMEM_SHARED((16 * NPAD,), jnp.float32),
        pltpu.SemaphoreType.DMA,
        pltpu.SemaphoreType.DMA,
    ],
)
def _deg_kernel(dst_hbm, deg_out, dstb, hist, comb, tmp, slots, sem0, sem1):
    c = lax.axis_index("c")
    s = lax.axis_index("s")
    w = c * jnp.int32(16) + s
    _zero_vmem(hist, NPAD)
    ones = jnp.ones((16,), jnp.float32)
    row0 = w * jnp.int32(SUPER * SCROWS)
    sems = (sem0, sem1)

    def issue(i, slot):
        r = pl.ds(row0 + jnp.int32(i * SCROWS), SCROWS)
        b = dstb.at[pl.ds(jnp.int32(slot * SCROWS), SCROWS), :]
        return pltpu.async_copy(dst_hbm.at[r, :], b, sems[slot])

    pend = [issue(0, 0), None]
    for i in range(SUPER):
        slot = i & 1
        if i + 1 < SUPER:
            pend[(i + 1) & 1] = issue(i + 1, (i + 1) & 1)
        pend[slot].wait()
        for j in range(SCROWS):
            r = jnp.int32(slot * SCROWS + j)
            for j2 in range(CHUNK // 16):
                idxv = dstb[r, pl.ds(j2 * 16, 16)]
                plsc.addupdate_scatter(hist, [idxv], ones)

    _publish_and_reduce(hist, slots, comb, tmp, s, deg_out,
                        c * jnp.int32(NPAD) + s * jnp.int32(SLICE))


@functools.partial(
    pl.kernel,
    mesh=_mesh,
    out_type=[
        jax.ShapeDtypeStruct((2 * NPAD,), jnp.float32),
        jax.ShapeDtypeStruct((NPAD,), jnp.float32),
        jax.ShapeDtypeStruct((NPAD,), jnp.float32),
    ],
    compiler_params=pltpu.CompilerParams(needs_layout_passes=False),
    scratch_types=[
        pltpu.VMEM((2 * SCROWS, CHUNK), jnp.int32),
        pltpu.VMEM((2 * SCROWS, CHUNK), jnp.int32),
        pltpu.VMEM((NPAD,), jnp.float32),
        pltpu.VMEM((SLICE,), jnp.float32),
        pltpu.VMEM((SLICE,), jnp.float32),
        pltpu.VMEM((SLICE,), jnp.float32),
        pltpu.VMEM((SLICE,), jnp.float32),
        pltpu.VMEM((NPAD,), jnp.float32),
        pltpu.VMEM_SHARED((16 * NPAD,), jnp.float32),
        pltpu.SemaphoreType.DMA,
        pltpu.SemaphoreType.DMA,
    ],
)
def _main_kernel(src_hbm, dst_hbm, h_hbm, deg_hbm,
                 acc_out, dinv_out, g_out,
                 srcb, dstb, hist, d0, gv, hv, dinvv, gall,
                 slots, sem0, sem1):
    c = lax.axis_index("c")
    s = lax.axis_index("s")
    w = c * jnp.int32(16) + s
    sl = s * jnp.int32(SLICE)
    pltpu.sync_copy(deg_hbm.at[pl.ds(sl, SLICE)], d0)
    pltpu.sync_copy(deg_hbm.at[pl.ds(jnp.int32(NPAD) + sl, SLICE)], gv)
    pltpu.sync_copy(h_hbm.at[pl.ds(sl, SLICE)], hv)
    for j in range(SLICE // 16):
        dsl = pl.ds(j * 16, 16)
        deg = d0[dsl] + gv[dsl] + 1.0
        iv = plsc.bitcast(deg, jnp.int32)
        y = plsc.bitcast(jnp.int32(0x5F3759DF) - (iv >> 1), jnp.float32)
        for _ in range(3):
            y = y * (1.5 - 0.5 * deg * y * y)
        dinvv[dsl] = y
        gv[dsl] = y * hv[dsl]
    pltpu.sync_copy(gv, slots.at[pl.ds(sl, SLICE)])

    @pl.when(c == 0)
    def _():
        pltpu.sync_copy(dinvv, dinv_out.at[pl.ds(sl, SLICE)])
        pltpu.sync_copy(gv, g_out.at[pl.ds(sl, SLICE)])

    plsc.subcore_barrier()
    pltpu.sync_copy(slots.at[pl.ds(jnp.int32(0), NPAD)], gall)
    plsc.subcore_barrier()
    _zero_vmem(hist, NPAD)
    row0 = w * jnp.int32(SUPER * SCROWS)
    sems = (sem0, sem1)

    def issue(i, slot):
        r = pl.ds(row0 + jnp.int32(i * SCROWS), SCROWS)
        bs = srcb.at[pl.ds(jnp.int32(slot * SCROWS), SCROWS), :]
        bd = dstb.at[pl.ds(jnp.int32(slot * SCROWS), SCROWS), :]
        return (pltpu.async_copy(src_hbm.at[r, :], bs, sems[slot]),
                pltpu.async_copy(dst_hbm.at[r, :], bd, sems[slot]))

    pend = [issue(0, 0), None]
    for i in range(SUPER):
        slot = i & 1
        if i + 1 < SUPER:
            pend[(i + 1) & 1] = issue(i + 1, (i + 1) & 1)
        ca, cb = pend[slot]
        ca.wait()
        cb.wait()
        for j in range(SCROWS):
            r = jnp.int32(slot * SCROWS + j)
            for j2 in range(CHUNK // 16):
                idxv = srcb[r, pl.ds(j2 * 16, 16)]
                vals = plsc.load_gather(gall, [idxv])
                didx = dstb[r, pl.ds(j2 * 16, 16)]
                plsc.addupdate_scatter(hist, [didx], vals)

    _publish_and_reduce(hist, slots, dinvv, hv, s, acc_out,
                        c * jnp.int32(NPAD) + sl)


def _epi_body(a_ref, dinv_ref, g_ref, o_ref):
    acc = a_ref[0] + a_ref[1]
    o = dinv_ref[...] * (acc + g_ref[...])
    t = jnp.exp(-jnp.abs(o))
    sp = jnp.maximum(o, 0.0) + jnp.log1p(t)
    o_ref[...] = 1.0 / (sp + TAU0)


_epilogue = pl.pallas_call(
    _epi_body,
    out_shape=jax.ShapeDtypeStruct((NPAD // D, D), jnp.float32),
)


def kernel(x, edge_index, edge_attr, W):
    del edge_attr
    x = x.astype(jnp.float32)
    W = W.astype(jnp.float32)
    ei = edge_index.astype(jnp.int32)
    src, dst = ei[0], ei[1]
    npad_e = EPAD - E
    spread = (jnp.arange(npad_e, dtype=jnp.int32) % (NPAD - N)) + N
    src_p = jnp.concatenate([src, spread]).reshape(EROWS, CHUNK)
    dst_p = jnp.concatenate([dst, spread]).reshape(EROWS, CHUNK)

    h = _h_matvec(x, W).reshape(NPAD)
    deg = _deg_kernel(dst_p)
    acc, dinv, g = _main_kernel(src_p, dst_p, h, deg)
    temp = _epilogue(
        acc.reshape(2, NPAD // D, D),
        dinv.reshape(NPAD // D, D),
        g.reshape(NPAD // D, D),
    )
    return temp.reshape(NPAD)[:N].reshape(N, 1)
